# baseline (speedup 1.0000x reference)
import functools

import jax
import jax.numpy as jnp
from jax import lax
from jax.experimental import pallas as pl
from jax.experimental.pallas import tpu as pltpu
from jax.experimental.pallas import tpu_sc as plsc

_LANES = 16
_CHUNK = 20000


@functools.cache
def _make_sc_kernel(n: int, chunk: int):
    num_chunks = n // chunk
    info = plsc.get_sparse_core_info()
    nc, ns = info.num_cores, info.num_subcores
    nw = nc * ns
    nvec = chunk // _LANES
    nbuf = 2
    max_j = -(-num_chunks // nw)
    max_j += (-max_j) % nbuf
    mesh = plsc.VectorSubcoreMesh(core_axis_name="c", subcore_axis_name="s")

    @functools.partial(
        pl.kernel,
        mesh=mesh,
        out_type=jax.ShapeDtypeStruct((n,), jnp.float32),
        scratch_types=[
            pltpu.VMEM((_LANES,), jnp.float32),
            pltpu.VMEM((_LANES,), jnp.float32),
            [pltpu.VMEM((chunk,), jnp.float32)] * nbuf,
            [pltpu.VMEM((chunk,), jnp.int32)] * nbuf,
            [pltpu.VMEM((chunk,), jnp.float32)] * nbuf,
            [pltpu.SemaphoreType.DMA] * nbuf,
            [pltpu.SemaphoreType.DMA] * nbuf,
            [pltpu.SemaphoreType.DMA] * nbuf,
        ],
    )
    def k(x_hbm, head_hbm, scale_hbm, shift_hbm, out_hbm,
          scale_v, shift_v, xv, hv, yv, xsems, hsems, ysems):
        wid = lax.axis_index("s") * nc + lax.axis_index("c")
        pltpu.sync_copy(scale_hbm, scale_v.at[pl.ds(0, 4)])
        pltpu.sync_copy(shift_hbm, shift_v.at[pl.ds(0, 4)])
        sv = scale_v[...]
        tv = shift_v[...]

        def in_copies(j, b):
            base = (wid + j * nw) * chunk
            return (
                pltpu.make_async_copy(
                    x_hbm.at[pl.ds(base, chunk)], xv[b], xsems[b]),
                pltpu.make_async_copy(
                    head_hbm.at[pl.ds(base, chunk)], hv[b], hsems[b]),
            )

        def out_copy(j, b):
            base = (wid + j * nw) * chunk
            return pltpu.make_async_copy(
                yv[b], out_hbm.at[pl.ds(base, chunk)], ysems[b])

        def valid(j):
            return wid + j * nw < num_chunks

        for b in range(nbuf):
            @pl.when(valid(b))
            def _(b=b):
                for c in in_copies(b, b):
                    c.start()

        @pl.loop(0, max_j, step=nbuf)
        def _(jj):
            for b in range(nbuf):
                j = jj + b

                @pl.when(valid(j))
                def _(j=j, b=b):
                    for c in in_copies(j, b):
                        c.wait()

                    @pl.when(j >= nbuf)
                    def _():
                        out_copy(j - nbuf, b).wait()

                    @plsc.parallel_loop(0, chunk, step=_LANES, unroll=4)
                    def _(off, b=b):
                        idx = hv[b][pl.ds(off, _LANES)]
                        s = sv.at[idx].get(mode="promise_in_bounds")
                        t = tv.at[idx].get(mode="promise_in_bounds")
                        yv[b][pl.ds(off, _LANES)] = (
                            s * xv[b][pl.ds(off, _LANES)] + t)

                    out_copy(j, b).start()

                    @pl.when(valid(j + nbuf))
                    def _():
                        for c in in_copies(j + nbuf, b):
                            c.start()

        my_j = (num_chunks - wid + nw - 1) // nw
        for b in range(nbuf):
            @pl.when(my_j >= b + 1)
            def _(b=b):
                jb = ((my_j - 1 - b) // nbuf) * nbuf + b
                out_copy(jb, b).wait()

    return k


def kernel(x, head, scale, shift):
    n = x.shape[0]
    head = head.astype(jnp.int32)
    scale = jnp.atleast_1d(scale).astype(jnp.float32)
    shift = jnp.atleast_1d(shift).astype(jnp.float32)
    return _make_sc_kernel(n, _CHUNK)(x, head, scale, shift)


# speedup vs baseline: 1.0579x; 1.0579x over previous
import functools

import jax
import jax.numpy as jnp
from jax import lax
from jax.experimental import pallas as pl
from jax.experimental.pallas import tpu as pltpu
from jax.experimental.pallas import tpu_sc as plsc

_LANES = 16
_CHUNK = 8000


@functools.cache
def _make_sc_kernel(n: int, chunk: int):
    num_chunks = n // chunk
    info = plsc.get_sparse_core_info()
    nc, ns = info.num_cores, info.num_subcores
    nw = nc * ns
    nvec = chunk // _LANES
    nbuf = 4
    max_j = -(-num_chunks // nw)
    max_j += (-max_j) % nbuf
    mesh = plsc.VectorSubcoreMesh(core_axis_name="c", subcore_axis_name="s")

    @functools.partial(
        pl.kernel,
        mesh=mesh,
        out_type=jax.ShapeDtypeStruct((n,), jnp.float32),
        scratch_types=[
            pltpu.VMEM((_LANES,), jnp.float32),
            pltpu.VMEM((_LANES,), jnp.float32),
            [pltpu.VMEM((chunk,), jnp.float32)] * nbuf,
            [pltpu.VMEM((chunk,), jnp.int32)] * nbuf,
            [pltpu.VMEM((chunk,), jnp.float32)] * nbuf,
            [pltpu.SemaphoreType.DMA] * nbuf,
            [pltpu.SemaphoreType.DMA] * nbuf,
            [pltpu.SemaphoreType.DMA] * nbuf,
        ],
    )
    def k(x_hbm, head_hbm, scale_hbm, shift_hbm, out_hbm,
          scale_v, shift_v, xv, hv, yv, xsems, hsems, ysems):
        wid = lax.axis_index("s") * nc + lax.axis_index("c")
        pltpu.sync_copy(scale_hbm, scale_v.at[pl.ds(0, 4)])
        pltpu.sync_copy(shift_hbm, shift_v.at[pl.ds(0, 4)])
        sv = scale_v[...]
        tv = shift_v[...]

        def in_copies(j, b):
            base = (wid + j * nw) * chunk
            return (
                pltpu.make_async_copy(
                    x_hbm.at[pl.ds(base, chunk)], xv[b], xsems[b]),
                pltpu.make_async_copy(
                    head_hbm.at[pl.ds(base, chunk)], hv[b], hsems[b]),
            )

        def out_copy(j, b):
            base = (wid + j * nw) * chunk
            return pltpu.make_async_copy(
                yv[b], out_hbm.at[pl.ds(base, chunk)], ysems[b])

        def valid(j):
            return wid + j * nw < num_chunks

        for b in range(nbuf):
            @pl.when(valid(b))
            def _(b=b):
                for c in in_copies(b, b):
                    c.start()

        @pl.loop(0, max_j, step=nbuf)
        def _(jj):
            for b in range(nbuf):
                j = jj + b

                @pl.when(valid(j))
                def _(j=j, b=b):
                    for c in in_copies(j, b):
                        c.wait()

                    @pl.when(j >= nbuf)
                    def _():
                        out_copy(j - nbuf, b).wait()

                    @plsc.parallel_loop(0, chunk, step=_LANES, unroll=4)
                    def _(off, b=b):
                        idx = hv[b][pl.ds(off, _LANES)]
                        s = sv.at[idx].get(mode="promise_in_bounds")
                        t = tv.at[idx].get(mode="promise_in_bounds")
                        yv[b][pl.ds(off, _LANES)] = (
                            s * xv[b][pl.ds(off, _LANES)] + t)

                    out_copy(j, b).start()

                    @pl.when(valid(j + nbuf))
                    def _():
                        for c in in_copies(j + nbuf, b):
                            c.start()

        my_j = (num_chunks - wid + nw - 1) // nw
        for b in range(nbuf):
            @pl.when(my_j >= b + 1)
            def _(b=b):
                jb = ((my_j - 1 - b) // nbuf) * nbuf + b
                out_copy(jb, b).wait()

    return k


def kernel(x, head, scale, shift):
    n = x.shape[0]
    head = head.astype(jnp.int32)
    scale = jnp.atleast_1d(scale).astype(jnp.float32)
    shift = jnp.atleast_1d(shift).astype(jnp.float32)
    return _make_sc_kernel(n, _CHUNK)(x, head, scale, shift)
